# Initial kernel scaffold; baseline (speedup 1.0000x reference)
import jax
import jax.numpy as jnp
from jax.experimental import pallas as pl


def kernel(x, W, b, codebook):
    raise NotImplementedError("write your pallas kernel here")


# speedup vs baseline: 1.9230x; 1.9230x over previous
import jax
import jax.numpy as jnp
from jax.experimental import pallas as pl


def _tc_body(x_ref, w_ref, b_ref, cb_ref, cbt_ref, q_ref, n_ref, i_ref):
    xb = x_ref[0]
    K, T = cb_ref.shape[0], xb.shape[1]
    z = jnp.dot(w_ref[...], xb, preferred_element_type=jnp.float32)
    z = z + b_ref[...][:, None]
    s = jnp.dot(cb_ref[...], z, preferred_element_type=jnp.float32)
    zz = jnp.sum(z * z, axis=0, keepdims=True)
    cc = jnp.sum(cb_ref[...] * cb_ref[...], axis=1, keepdims=True)
    d2 = (zz - 2.0 * s) + cc
    m = jnp.min(d2, axis=0, keepdims=True)
    kio = jax.lax.broadcasted_iota(jnp.int32, (K, T), 0)
    idx = jnp.min(jnp.where(d2 == m, kio, K), axis=0, keepdims=True)
    oh = (kio == idx).astype(jnp.float32)
    q = jax.lax.dot_general(cbt_ref[...], oh, (((1,), (0,)), ((), ())),
                            precision=jax.lax.Precision.HIGHEST,
                            preferred_element_type=jnp.float32)
    q_ref[0] = q
    d = z - q
    n_ref[0] = jnp.sum(d * d, axis=0, keepdims=True)
    i_ref[0] = idx


def kernel(x, W, b, codebook):
    B, C_IN, T = x.shape
    C_OUT = W.shape[0]
    K = codebook.shape[0]
    cbT = codebook.T
    q, n, _idx = pl.pallas_call(
        _tc_body,
        grid=(B,),
        in_specs=[
            pl.BlockSpec((1, C_IN, T), lambda b_: (b_, 0, 0)),
            pl.BlockSpec((C_OUT, C_IN), lambda b_: (0, 0)),
            pl.BlockSpec((C_OUT,), lambda b_: (0,)),
            pl.BlockSpec((K, C_OUT), lambda b_: (0, 0)),
            pl.BlockSpec((C_OUT, K), lambda b_: (0, 0)),
        ],
        out_specs=[
            pl.BlockSpec((1, C_OUT, T), lambda b_: (b_, 0, 0)),
            pl.BlockSpec((1, 1, T), lambda b_: (b_, 0, 0)),
            pl.BlockSpec((1, 1, T), lambda b_: (b_, 0, 0)),
        ],
        out_shape=[
            jax.ShapeDtypeStruct((B, C_OUT, T), jnp.float32),
            jax.ShapeDtypeStruct((B, 1, T), jnp.float32),
            jax.ShapeDtypeStruct((B, 1, T), jnp.int32),
        ],
    )(x, W, b, codebook, cbT)
    n = n.reshape(B, T)
    vq_norms = jnp.stack([n, n], axis=-1)
    return q, vq_norms
